# Initial kernel scaffold; baseline (speedup 1.0000x reference)
import jax
import jax.numpy as jnp
from jax import lax
from jax.experimental import pallas as pl

TOKENS = 32768
EMB_DIM = 768
NUM_EXPERTS = 64
BM = 512


def _gate_kernel(h_ref, wt_ref, b_ref, out_ref):
    g = jnp.dot(h_ref[...], wt_ref[...], preferred_element_type=jnp.float32)
    g = g + b_ref[...]
    m1 = jnp.max(g, axis=1, keepdims=True)
    col = lax.broadcasted_iota(jnp.int32, g.shape, 1)
    idx1 = jnp.min(jnp.where(g == m1, col, NUM_EXPERTS), axis=1, keepdims=True)
    g_wo_max = jnp.where(col == idx1, -jnp.inf, g)
    m2 = jnp.max(g_wo_max, axis=1, keepdims=True)
    mask = g >= m2
    e = jnp.where(mask, jnp.exp(g - m1), 0.0)
    out_ref[...] = e / jnp.sum(e, axis=1, keepdims=True)


@jax.jit
def kernel(h, W, b):
    wt = W.T
    b2 = b.reshape(1, NUM_EXPERTS)
    grid = (TOKENS // BM,)
    return pl.pallas_call(
        _gate_kernel,
        grid=grid,
        in_specs=[
            pl.BlockSpec((BM, EMB_DIM), lambda i: (i, 0)),
            pl.BlockSpec((EMB_DIM, NUM_EXPERTS), lambda i: (0, 0)),
            pl.BlockSpec((1, NUM_EXPERTS), lambda i: (0, 0)),
        ],
        out_specs=pl.BlockSpec((BM, NUM_EXPERTS), lambda i: (i, 0)),
        out_shape=jax.ShapeDtypeStruct((TOKENS, NUM_EXPERTS), jnp.float32),
    )(h, wt, b2)


# speedup vs baseline: 8.7166x; 8.7166x over previous
import jax
import jax.numpy as jnp
from jax import lax
from jax.experimental import pallas as pl

TOKENS = 32768
EMB_DIM = 768
NUM_EXPERTS = 64
BM = 4096


def _gate_kernel(h_ref, wt_ref, b_ref, out_ref):
    g = jnp.dot(h_ref[...], wt_ref[...], preferred_element_type=jnp.float32)
    g = g + b_ref[...]
    m1 = jnp.max(g, axis=1, keepdims=True)
    m2 = jnp.max(jnp.where(g == m1, -jnp.inf, g), axis=1, keepdims=True)
    e = jnp.where(g >= m2, jnp.exp(g - m1), 0.0)
    out_ref[...] = e / jnp.sum(e, axis=1, keepdims=True)


@jax.jit
def kernel(h, W, b):
    wt = W.T
    b2 = b.reshape(1, NUM_EXPERTS)
    grid = (TOKENS // BM,)
    return pl.pallas_call(
        _gate_kernel,
        grid=grid,
        in_specs=[
            pl.BlockSpec((BM, EMB_DIM), lambda i: (i, 0)),
            pl.BlockSpec((EMB_DIM, NUM_EXPERTS), lambda i: (0, 0)),
            pl.BlockSpec((1, NUM_EXPERTS), lambda i: (0, 0)),
        ],
        out_specs=pl.BlockSpec((BM, NUM_EXPERTS), lambda i: (i, 0)),
        out_shape=jax.ShapeDtypeStruct((TOKENS, NUM_EXPERTS), jnp.float32),
    )(h, wt, b2)
